# baseline (speedup 1.0000x reference)
import jax
import jax.numpy as jnp
from jax.experimental import pallas as pl

BM = 128


def _rbf_block(x_ref, yt_ref, o_ref):
    xb = x_ref[...]
    yb = yt_ref[...]
    x_hi = xb.astype(jnp.bfloat16)
    x_lo = (xb - x_hi.astype(jnp.float32)).astype(jnp.bfloat16)
    y_hi = yb.astype(jnp.bfloat16)
    y_lo = (yb - y_hi.astype(jnp.float32)).astype(jnp.bfloat16)
    lhs = jnp.concatenate([x_hi, x_hi, x_lo], axis=1)
    rhs = jnp.concatenate([y_hi, y_lo, y_hi], axis=0)
    z = jnp.dot(lhs, rhs, preferred_element_type=jnp.float32)
    xn = jnp.sum(xb * xb, axis=1, keepdims=True)
    yn = jnp.sum(yb * yb, axis=0, keepdims=True)
    o_ref[...] = jnp.exp(z - 0.5 * (xn + yn))


def kernel(x, y):
    n_row, d = x.shape
    n_col = y.shape[0]
    yt = y.T
    grid = (n_row // BM,)
    return pl.pallas_call(
        _rbf_block,
        grid=grid,
        in_specs=[
            pl.BlockSpec((BM, d), lambda i: (i, 0)),
            pl.BlockSpec((d, n_col), lambda i: (0, 0)),
        ],
        out_specs=pl.BlockSpec((BM, n_col), lambda i: (i, 0)),
        out_shape=jax.ShapeDtypeStruct((n_row, n_col), jnp.float32),
    )(x, yt)


# speedup vs baseline: 1.5023x; 1.5023x over previous
import jax
import jax.numpy as jnp
from jax.experimental import pallas as pl

BM = 512


def _rbf_block(x_ref, yt_ref, o_ref):
    xb = x_ref[...]
    yb = yt_ref[...]
    x_hi = xb.astype(jnp.bfloat16)
    x_lo = (xb - x_hi.astype(jnp.float32)).astype(jnp.bfloat16)
    y_hi = yb.astype(jnp.bfloat16)
    y_lo = (yb - y_hi.astype(jnp.float32)).astype(jnp.bfloat16)
    lhs = jnp.concatenate([x_hi, x_hi, x_lo], axis=1)
    rhs = jnp.concatenate([y_hi, y_lo, y_hi], axis=0)
    z = jnp.dot(lhs, rhs, preferred_element_type=jnp.float32)
    xn = jnp.sum(xb * xb, axis=1, keepdims=True)
    yn = jnp.sum(yb * yb, axis=0, keepdims=True)
    o_ref[...] = jnp.exp(z - 0.5 * (xn + yn))


def kernel(x, y):
    n_row, d = x.shape
    n_col = y.shape[0]
    yt = y.T
    grid = (n_row // BM,)
    return pl.pallas_call(
        _rbf_block,
        grid=grid,
        in_specs=[
            pl.BlockSpec((BM, d), lambda i: (i, 0)),
            pl.BlockSpec((d, n_col), lambda i: (0, 0)),
        ],
        out_specs=pl.BlockSpec((BM, n_col), lambda i: (i, 0)),
        out_shape=jax.ShapeDtypeStruct((n_row, n_col), jnp.float32),
    )(x, yt)
